# Initial kernel scaffold; baseline (speedup 1.0000x reference)
import jax
import jax.numpy as jnp
from jax.experimental import pallas as pl


def kernel(board_tensor, emb_table):
    raise NotImplementedError("write your pallas kernel here")


# speedup vs baseline: 3.2796x; 3.2796x over previous
import jax
import jax.numpy as jnp
from jax.experimental import pallas as pl


def _body(board_ref, emb_ref, out_ref):
    bb, s, f = board_ref.shape
    e = emb_ref.shape[1]
    out_ref[:, :, :f] = board_ref[...]
    out_ref[:, :, f:] = jnp.broadcast_to(emb_ref[...][None, :, :], (bb, s, e))


def kernel(board_tensor, emb_table):
    B, S, F = board_tensor.shape
    E = emb_table.shape[1]
    BB = 64
    return pl.pallas_call(
        _body,
        grid=(B // BB,),
        in_specs=[
            pl.BlockSpec((BB, S, F), lambda i: (i, 0, 0)),
            pl.BlockSpec((S, E), lambda i: (0, 0)),
        ],
        out_specs=pl.BlockSpec((BB, S, F + E), lambda i: (i, 0, 0)),
        out_shape=jax.ShapeDtypeStruct((B, S, F + E), jnp.float32),
    )(board_tensor, emb_table)


# speedup vs baseline: 3.3580x; 1.0239x over previous
import jax
import jax.numpy as jnp
from jax.experimental import pallas as pl


def _body(board_ref, emb_ref, out_ref):
    bb, s, f = board_ref.shape
    e = emb_ref.shape[1]
    out_ref[:, :, :f] = board_ref[...]
    out_ref[:, :, f:] = jnp.broadcast_to(emb_ref[...][None, :, :], (bb, s, e))


def kernel(board_tensor, emb_table):
    B, S, F = board_tensor.shape
    E = emb_table.shape[1]
    BB = 256
    return pl.pallas_call(
        _body,
        grid=(B // BB,),
        in_specs=[
            pl.BlockSpec((BB, S, F), lambda i: (i, 0, 0)),
            pl.BlockSpec((S, E), lambda i: (0, 0)),
        ],
        out_specs=pl.BlockSpec((BB, S, F + E), lambda i: (i, 0, 0)),
        out_shape=jax.ShapeDtypeStruct((B, S, F + E), jnp.float32),
    )(board_tensor, emb_table)
